# baseline (speedup 1.0000x reference)
import functools

import jax
import jax.numpy as jnp
from jax import lax
from jax.experimental import pallas as pl
from jax.experimental.pallas import tpu as pltpu
from jax.experimental.pallas import tpu_sc as plsc

N_NODES = 51200
N_GRAPHS = 128
N_LABELS = 256
D = 256
M_EDGES = N_GRAPHS * N_LABELS
NCAT = N_GRAPHS + N_LABELS

NC, NS, L = 2, 16, 16
NW = NC * NS
E_PER_W = N_NODES // NW
NVREG = E_PER_W // L
ACC_ROWS = 144
M_PER_W = M_EDGES // NW

_mesh = lambda: plsc.VectorSubcoreMesh(
    core_axis_name="c", subcore_axis_name="s", num_cores=NC, num_subcores=NS)


ACC_FLAT = (N_GRAPHS + 1) * D
BMP_WORDS = N_NODES // 32


def _sc_accumulate(x, src, dst, targets):
  @functools.partial(
      pl.kernel,
      out_type=jax.ShapeDtypeStruct((NW, N_GRAPHS * D), jnp.float32),
      mesh=_mesh(),
      compiler_params=pltpu.CompilerParams(needs_layout_passes=False),
      scratch_types=[
          pltpu.VMEM((E_PER_W,), jnp.int32),
          pltpu.VMEM((E_PER_W,), jnp.int32),
          pltpu.VMEM((N_GRAPHS,), jnp.int32),
          pltpu.VMEM((E_PER_W + L,), jnp.int32),
          pltpu.VMEM((E_PER_W + L,), jnp.int32),
          pltpu.VMEM((L,), jnp.int32),
          pltpu.VMEM((L, D), jnp.float32),
          pltpu.VMEM((ACC_FLAT,), jnp.float32),
          pltpu.VMEM((BMP_WORDS,), jnp.int32),
          pltpu.SemaphoreType.DMA,
      ],
  )
  def k(x_hbm, src_hbm, dst_hbm, tgt_hbm, out_hbm,
        dst_v, src_v, tgt_v, csrc_v, cdst_v, idx16_v, rows_v, acc_v, bmp_v,
        sem):
    c = lax.axis_index("c")
    s = lax.axis_index("s")
    wid = c * NS + s
    base = wid * E_PER_W
    lane = lax.iota(jnp.int32, L)
    zero16 = jnp.zeros((L,), jnp.float32)
    one16 = jnp.ones((L,), jnp.int32)

    def zero_body(i, _):
      for u in range(16):
        acc_v[pl.ds(i * (16 * L) + u * L, L)] = zero16
      return 0

    lax.fori_loop(0, ACC_FLAT // (16 * L), zero_body, 0)
    for u in range(ACC_FLAT % (16 * L) // L):
      acc_v[pl.ds((ACC_FLAT // (16 * L)) * 16 * L + u * L, L)] = zero16

    def zbmp_body(i, _):
      for u in range(10):
        bmp_v[pl.ds(i * (10 * L) + u * L, L)] = jnp.zeros((L,), jnp.int32)
      return 0

    lax.fori_loop(0, BMP_WORDS // (10 * L), zbmp_body, 0)

    pltpu.sync_copy(dst_hbm.at[pl.ds(base, E_PER_W)], dst_v)
    pltpu.sync_copy(src_hbm.at[pl.ds(base, E_PER_W)], src_v)
    pltpu.sync_copy(tgt_hbm, tgt_v)

    for v in range(N_GRAPHS // L):
      idx = lane + v * L
      t = tgt_v[pl.ds(v * L, L)]
      prev = plsc.load_gather(tgt_v, [jnp.maximum(idx - 1, 0)])
      fresh = (t != prev) | (idx == 0)
      bit = jnp.left_shift(one16, jnp.bitwise_and(t, 31))
      plsc.addupdate_scatter(bmp_v, [jnp.right_shift(t, 5)], bit, mask=fresh)

    lane_base = lane * NVREG

    def scan_body(i, kvec):
      upd = kvec
      for u in range(2):
        off = (2 * i + u) * L
        dstv = dst_v[pl.ds(off, L)]
        srcv = src_v[pl.ds(off, L)]
        word = plsc.load_gather(bmp_v, [jnp.right_shift(dstv, 5)])
        m = jnp.bitwise_and(
            jnp.right_shift(word, jnp.bitwise_and(dstv, 31)), 1) == 1
        pos = lane_base + upd
        plsc.store_scatter(csrc_v, [pos], srcv, mask=m)
        plsc.store_scatter(cdst_v, [pos], dstv, mask=m)
        upd = upd + m.astype(jnp.int32)
      return upd

    kvec = lax.fori_loop(0, NVREG // 2, scan_body, jnp.zeros((L,), jnp.int32))
    csrc_v[pl.ds(E_PER_W, L)] = kvec
    nch = jnp.minimum(jnp.sum(kvec), NVREG)

    def acc_body(j, _):
      valid = j < csrc_v[pl.ds(E_PER_W, L)]
      pos = lane_base + j
      idxv = jnp.where(valid, plsc.load_gather(csrc_v, [pos]), 0)
      dstv = plsc.load_gather(cdst_v, [pos])
      lo = jnp.zeros((L,), jnp.int32)
      for h in (64, 32, 16, 8, 4, 2, 1):
        tv = plsc.load_gather(tgt_v, [lo + (h - 1)])
        lo = jnp.where(tv < dstv, lo + h, lo)
      slotv = jnp.where(valid, lo, N_GRAPHS)
      rowbase = slotv * D
      pltpu.async_copy(x_hbm.at[idxv], rows_v, sem).wait()
      for col in range(D):
        vals = plsc.load_gather(rows_v, [lane, jnp.full((L,), col, jnp.int32)])
        plsc.addupdate_scatter(acc_v, [rowbase + col], vals)
      return 0

    lax.fori_loop(0, nch, acc_body, 0)

    pltpu.sync_copy(acc_v.at[pl.ds(0, N_GRAPHS * D)], out_hbm.at[wid])

  return k(x, src, dst, targets)


def _tc_gram(partials, dst_r, tgt_col, tgt_row, x_label, scale_half):
  n_rows = dst_r.shape[0]

  def body(part_ref, dst_ref, tcol_ref, trow_ref, xl_ref, sh_ref, g_ref,
           xcat_ref):
    tcol = tcol_ref[...]

    def cnt_body(i, cnt):
      row = dst_ref[pl.ds(i, 1), :]
      eq = (tcol == row).astype(jnp.float32)
      return cnt + jnp.sum(eq, axis=1, keepdims=True)

    cnt = lax.fori_loop(0, n_rows, cnt_body,
                        jnp.zeros((N_GRAPHS, 1), jnp.float32))
    sums = jnp.sum(part_ref[...], axis=0)
    mean = sums / jnp.maximum(cnt, 1.0)

    eq2 = tcol == trow_ref[...]
    iota_col = lax.broadcasted_iota(jnp.int32, (N_GRAPHS, N_GRAPHS), 1)
    s_g = jnp.min(jnp.where(eq2, iota_col, N_GRAPHS), axis=1, keepdims=True)
    dmat = (iota_col == s_g).astype(jnp.float32)
    mean_full = lax.dot_general(dmat, mean, (((1,), (0,)), ((), ())),
                                preferred_element_type=jnp.float32,
                                precision=lax.Precision.HIGHEST)

    xcat_ref[pl.ds(0, N_GRAPHS), :] = mean_full
    xcat_ref[pl.ds(N_GRAPHS, N_LABELS), :] = xl_ref[...]
    xc = xcat_ref[...]
    nrm = jnp.sqrt(jnp.sum(xc * xc, axis=1, keepdims=True))
    xn = (xc / jnp.maximum(nrm, 1e-8)) * sh_ref[...]
    g_ref[...] = lax.dot_general(xn, xn, (((1,), (1,)), ((), ())),
                                 preferred_element_type=jnp.float32,
                                 precision=lax.Precision.HIGHEST)

  return pl.pallas_call(
      body,
      out_shape=jax.ShapeDtypeStruct((NCAT, NCAT), jnp.float32),
      scratch_shapes=[pltpu.VMEM((NCAT, D), jnp.float32)],
  )(partials, dst_r, tgt_col, tgt_row, x_label, scale_half)


def _sc_decode(g16, mg0, mg1):
  @functools.partial(
      pl.kernel,
      out_type=jax.ShapeDtypeStruct((M_EDGES,), jnp.float32),
      mesh=_mesh(),
      compiler_params=pltpu.CompilerParams(needs_layout_passes=False),
      scratch_types=[
          pltpu.VMEM((M_PER_W,), jnp.int32),
          pltpu.VMEM((M_PER_W,), jnp.int32),
          pltpu.VMEM((128,), jnp.int32),
          pltpu.VMEM((128, 128), jnp.float32),
          pltpu.VMEM((M_PER_W,), jnp.float32),
          pltpu.SemaphoreType.DMA,
      ],
  )
  def k(g_hbm, mg0_hbm, mg1_hbm, out_hbm,
        mg0_v, mg1_v, q_v, rows_v, out_v, sem):
    c = lax.axis_index("c")
    s = lax.axis_index("s")
    wid = c * NS + s
    base = wid * M_PER_W
    lane = lax.iota(jnp.int32, L)

    pltpu.sync_copy(mg0_hbm.at[pl.ds(base, M_PER_W)], mg0_v)
    pltpu.sync_copy(mg1_hbm.at[pl.ds(base, M_PER_W)], mg1_v)

    for sub in range(M_PER_W // 128):
      for v in range(128 // L):
        off = sub * 128 + v * L
        flat = mg0_v[pl.ds(off, L)] * NCAT + mg1_v[pl.ds(off, L)]
        q_v[pl.ds(v * L, L)] = jnp.right_shift(flat, 7)
      pltpu.async_copy(g_hbm.at[q_v], rows_v, sem).wait()
      for v in range(128 // L):
        off = sub * 128 + v * L
        flat = mg0_v[pl.ds(off, L)] * NCAT + mg1_v[pl.ds(off, L)]
        r = jnp.bitwise_and(flat, 127)
        out_v[pl.ds(off, L)] = plsc.load_gather(rows_v, [lane + v * L, r])
    pltpu.sync_copy(out_v, out_hbm.at[pl.ds(base, M_PER_W)])

  return k(g16, mg0, mg1)


def kernel(x, supernode, ptr, edge_index_supernode, x_label, y_true_matrix,
           metagraph_edge_index, metagraph_edge_attr, query_set_mask,
           logit_scale):
  src = edge_index_supernode[0]
  dst = edge_index_supernode[1]
  targets = supernode + ptr[:N_GRAPHS]

  partials = _sc_accumulate(x, src, dst, targets).reshape(NW, N_GRAPHS, D)

  scale_half = jnp.exp(logit_scale * 0.5).reshape(1, 1)
  g = _tc_gram(partials, dst.reshape(-1, 1024), targets.reshape(N_GRAPHS, 1),
               targets.reshape(1, N_GRAPHS), x_label, scale_half)

  decoded = _sc_decode(g.reshape(-1, 128), metagraph_edge_index[0],
                       metagraph_edge_index[1])
  y_pred = decoded.reshape(N_GRAPHS, N_LABELS)
  return (y_true_matrix, y_pred)


# speedup vs baseline: 1.5107x; 1.5107x over previous
import functools

import jax
import jax.numpy as jnp
from jax import lax
from jax.experimental import pallas as pl
from jax.experimental.pallas import tpu as pltpu
from jax.experimental.pallas import tpu_sc as plsc

N_NODES = 51200
N_GRAPHS = 128
N_LABELS = 256
D = 256
M_EDGES = N_GRAPHS * N_LABELS
NCAT = N_GRAPHS + N_LABELS

NC, NS, L = 2, 16, 16
NW = NC * NS
E_PER_W = N_NODES // NW
NVREG = E_PER_W // L
ACC_ROWS = 144
M_PER_W = M_EDGES // NW

_mesh = lambda: plsc.VectorSubcoreMesh(
    core_axis_name="c", subcore_axis_name="s", num_cores=NC, num_subcores=NS)


ACC_FLAT = (N_GRAPHS + 1) * D
BMP_WORDS = N_NODES // 32


def _sc_accumulate(x, src, dst, targets):
  @functools.partial(
      pl.kernel,
      out_type=jax.ShapeDtypeStruct((NW, N_GRAPHS * D), jnp.float32),
      mesh=_mesh(),
      compiler_params=pltpu.CompilerParams(needs_layout_passes=False),
      scratch_types=[
          pltpu.VMEM((E_PER_W,), jnp.int32),
          pltpu.VMEM((E_PER_W,), jnp.int32),
          pltpu.VMEM((N_GRAPHS,), jnp.int32),
          pltpu.VMEM((E_PER_W + L,), jnp.int32),
          pltpu.VMEM((E_PER_W + L,), jnp.int32),
          pltpu.VMEM((L,), jnp.int32),
          pltpu.VMEM((L, D), jnp.float32),
          pltpu.VMEM((ACC_FLAT,), jnp.float32),
          pltpu.VMEM((BMP_WORDS,), jnp.int32),
          pltpu.SemaphoreType.DMA,
      ],
  )
  def k(x_hbm, src_hbm, dst_hbm, tgt_hbm, out_hbm,
        dst_v, src_v, tgt_v, csrc_v, cdst_v, idx16_v, rows_v, acc_v, bmp_v,
        sem):
    c = lax.axis_index("c")
    s = lax.axis_index("s")
    wid = c * NS + s
    base = wid * E_PER_W
    lane = lax.iota(jnp.int32, L)
    zero16 = jnp.zeros((L,), jnp.float32)
    one16 = jnp.ones((L,), jnp.int32)

    def zero_body(i, _):
      for u in range(16):
        acc_v[pl.ds(i * (16 * L) + u * L, L)] = zero16
      return 0

    lax.fori_loop(0, ACC_FLAT // (16 * L), zero_body, 0)
    for u in range(ACC_FLAT % (16 * L) // L):
      acc_v[pl.ds((ACC_FLAT // (16 * L)) * 16 * L + u * L, L)] = zero16

    def zbmp_body(i, _):
      for u in range(10):
        bmp_v[pl.ds(i * (10 * L) + u * L, L)] = jnp.zeros((L,), jnp.int32)
      return 0

    lax.fori_loop(0, BMP_WORDS // (10 * L), zbmp_body, 0)

    pltpu.sync_copy(dst_hbm.at[pl.ds(base, E_PER_W)], dst_v)
    pltpu.sync_copy(src_hbm.at[pl.ds(base, E_PER_W)], src_v)
    pltpu.sync_copy(tgt_hbm, tgt_v)

    for v in range(N_GRAPHS // L):
      idx = lane + v * L
      t = tgt_v[pl.ds(v * L, L)]
      prev = plsc.load_gather(tgt_v, [jnp.maximum(idx - 1, 0)])
      fresh = (t != prev) | (idx == 0)
      bit = jnp.left_shift(one16, jnp.bitwise_and(t, 31))
      plsc.addupdate_scatter(bmp_v, [jnp.right_shift(t, 5)], bit, mask=fresh)

    lane_base = lane * NVREG

    def scan_body(i, kvec):
      upd = kvec
      for u in range(2):
        off = (2 * i + u) * L
        dstv = dst_v[pl.ds(off, L)]
        srcv = src_v[pl.ds(off, L)]
        word = plsc.load_gather(bmp_v, [jnp.right_shift(dstv, 5)])
        m = jnp.bitwise_and(
            jnp.right_shift(word, jnp.bitwise_and(dstv, 31)), 1) == 1
        pos = lane_base + upd
        plsc.store_scatter(csrc_v, [pos], srcv, mask=m)
        plsc.store_scatter(cdst_v, [pos], dstv, mask=m)
        upd = upd + m.astype(jnp.int32)
      return upd

    kvec = lax.fori_loop(0, NVREG // 2, scan_body, jnp.zeros((L,), jnp.int32))
    csrc_v[pl.ds(E_PER_W, L)] = kvec
    nch = jnp.max(kvec)

    def acc_body(j, _):
      valid = j < csrc_v[pl.ds(E_PER_W, L)]
      pos = lane_base + j
      idxv = jnp.where(valid, plsc.load_gather(csrc_v, [pos]), 0)
      dstv = plsc.load_gather(cdst_v, [pos])
      lo = jnp.zeros((L,), jnp.int32)
      for h in (64, 32, 16, 8, 4, 2, 1):
        tv = plsc.load_gather(tgt_v, [lo + (h - 1)])
        lo = jnp.where(tv < dstv, lo + h, lo)
      slotv = jnp.where(valid, lo, N_GRAPHS)
      rowbase = slotv * D
      pltpu.async_copy(x_hbm.at[idxv], rows_v, sem).wait()
      for col in range(D):
        vals = plsc.load_gather(rows_v, [lane, jnp.full((L,), col, jnp.int32)])
        plsc.addupdate_scatter(acc_v, [rowbase + col], vals)
      return 0

    lax.fori_loop(0, nch, acc_body, 0)

    pltpu.sync_copy(acc_v.at[pl.ds(0, N_GRAPHS * D)], out_hbm.at[wid])

  return k(x, src, dst, targets)


def _tc_gram(partials, dst_r, tgt_col, tgt_row, x_label, scale_half):
  n_rows = dst_r.shape[0]

  def body(part_ref, dst_ref, tcol_ref, trow_ref, xl_ref, sh_ref, g_ref,
           xcat_ref):
    tcol = tcol_ref[...]

    def cnt_body(i, cnt):
      row = dst_ref[pl.ds(i, 1), :]
      eq = (tcol == row).astype(jnp.float32)
      return cnt + jnp.sum(eq, axis=1, keepdims=True)

    cnt = lax.fori_loop(0, n_rows, cnt_body,
                        jnp.zeros((N_GRAPHS, 1), jnp.float32))
    sums = jnp.sum(part_ref[...], axis=0)
    mean = sums / jnp.maximum(cnt, 1.0)

    eq2 = tcol == trow_ref[...]
    iota_col = lax.broadcasted_iota(jnp.int32, (N_GRAPHS, N_GRAPHS), 1)
    s_g = jnp.min(jnp.where(eq2, iota_col, N_GRAPHS), axis=1, keepdims=True)
    dmat = (iota_col == s_g).astype(jnp.float32)
    mean_full = lax.dot_general(dmat, mean, (((1,), (0,)), ((), ())),
                                preferred_element_type=jnp.float32,
                                precision=lax.Precision.HIGHEST)

    xcat_ref[pl.ds(0, N_GRAPHS), :] = mean_full
    xcat_ref[pl.ds(N_GRAPHS, N_LABELS), :] = xl_ref[...]
    xc = xcat_ref[...]
    nrm = jnp.sqrt(jnp.sum(xc * xc, axis=1, keepdims=True))
    xn = (xc / jnp.maximum(nrm, 1e-8)) * sh_ref[...]
    g_ref[...] = lax.dot_general(xn, xn, (((1,), (1,)), ((), ())),
                                 preferred_element_type=jnp.float32,
                                 precision=lax.Precision.HIGHEST)

  return pl.pallas_call(
      body,
      out_shape=jax.ShapeDtypeStruct((NCAT, NCAT), jnp.float32),
      scratch_shapes=[pltpu.VMEM((NCAT, D), jnp.float32)],
  )(partials, dst_r, tgt_col, tgt_row, x_label, scale_half)


def _sc_decode(g16, mg0, mg1):
  @functools.partial(
      pl.kernel,
      out_type=jax.ShapeDtypeStruct((M_EDGES,), jnp.float32),
      mesh=_mesh(),
      compiler_params=pltpu.CompilerParams(needs_layout_passes=False),
      scratch_types=[
          pltpu.VMEM((M_PER_W,), jnp.int32),
          pltpu.VMEM((M_PER_W,), jnp.int32),
          pltpu.VMEM((128,), jnp.int32),
          pltpu.VMEM((128, 128), jnp.float32),
          pltpu.VMEM((M_PER_W,), jnp.float32),
          pltpu.SemaphoreType.DMA,
      ],
  )
  def k(g_hbm, mg0_hbm, mg1_hbm, out_hbm,
        mg0_v, mg1_v, q_v, rows_v, out_v, sem):
    c = lax.axis_index("c")
    s = lax.axis_index("s")
    wid = c * NS + s
    base = wid * M_PER_W
    lane = lax.iota(jnp.int32, L)

    pltpu.sync_copy(mg0_hbm.at[pl.ds(base, M_PER_W)], mg0_v)
    pltpu.sync_copy(mg1_hbm.at[pl.ds(base, M_PER_W)], mg1_v)

    for sub in range(M_PER_W // 128):
      for v in range(128 // L):
        off = sub * 128 + v * L
        flat = mg0_v[pl.ds(off, L)] * NCAT + mg1_v[pl.ds(off, L)]
        q_v[pl.ds(v * L, L)] = jnp.right_shift(flat, 7)
      pltpu.async_copy(g_hbm.at[q_v], rows_v, sem).wait()
      for v in range(128 // L):
        off = sub * 128 + v * L
        flat = mg0_v[pl.ds(off, L)] * NCAT + mg1_v[pl.ds(off, L)]
        r = jnp.bitwise_and(flat, 127)
        out_v[pl.ds(off, L)] = plsc.load_gather(rows_v, [lane + v * L, r])
    pltpu.sync_copy(out_v, out_hbm.at[pl.ds(base, M_PER_W)])

  return k(g16, mg0, mg1)


def kernel(x, supernode, ptr, edge_index_supernode, x_label, y_true_matrix,
           metagraph_edge_index, metagraph_edge_attr, query_set_mask,
           logit_scale):
  src = edge_index_supernode[0]
  dst = edge_index_supernode[1]
  targets = supernode + ptr[:N_GRAPHS]

  partials = _sc_accumulate(x, src, dst, targets).reshape(NW, N_GRAPHS, D)

  scale_half = jnp.exp(logit_scale * 0.5).reshape(1, 1)
  g = _tc_gram(partials, dst.reshape(-1, 1024), targets.reshape(N_GRAPHS, 1),
               targets.reshape(1, N_GRAPHS), x_label, scale_half)

  decoded = _sc_decode(g.reshape(-1, 128), metagraph_edge_index[0],
                       metagraph_edge_index[1])
  y_pred = decoded.reshape(N_GRAPHS, N_LABELS)
  return (y_true_matrix, y_pred)


# speedup vs baseline: 2.1270x; 1.4080x over previous
import functools

import jax
import jax.numpy as jnp
from jax import lax
from jax.experimental import pallas as pl
from jax.experimental.pallas import tpu as pltpu
from jax.experimental.pallas import tpu_sc as plsc

N_NODES = 51200
N_GRAPHS = 128
N_LABELS = 256
D = 256
M_EDGES = N_GRAPHS * N_LABELS
NCAT = N_GRAPHS + N_LABELS

NC, NS, L = 2, 16, 16
NW = NC * NS
E_PER_W = N_NODES // NW
NVREG = E_PER_W // L
ACC_ROWS = 144
M_PER_W = M_EDGES // NW

_mesh = lambda: plsc.VectorSubcoreMesh(
    core_axis_name="c", subcore_axis_name="s", num_cores=NC, num_subcores=NS)


ACC_FLAT = (N_GRAPHS + 1) * D
BMP_WORDS = N_NODES // 32


def _sc_accumulate(x, src, dst, targets):
  @functools.partial(
      pl.kernel,
      out_type=jax.ShapeDtypeStruct((NW, N_GRAPHS * D), jnp.float32),
      mesh=_mesh(),
      compiler_params=pltpu.CompilerParams(needs_layout_passes=False),
      scratch_types=[
          pltpu.VMEM((E_PER_W,), jnp.int32),
          pltpu.VMEM((E_PER_W,), jnp.int32),
          pltpu.VMEM((N_GRAPHS,), jnp.int32),
          pltpu.VMEM((E_PER_W + L,), jnp.int32),
          pltpu.VMEM((E_PER_W + L,), jnp.int32),
          pltpu.VMEM((L,), jnp.int32),
          pltpu.VMEM((L, D), jnp.float32),
          pltpu.VMEM((ACC_FLAT,), jnp.float32),
          pltpu.VMEM((BMP_WORDS,), jnp.int32),
          pltpu.SemaphoreType.DMA,
      ],
  )
  def k(x_hbm, src_hbm, dst_hbm, tgt_hbm, out_hbm,
        dst_v, src_v, tgt_v, csrc_v, cdst_v, slot16_v, rows_v, acc_v, bmp_v,
        sem):
    c = lax.axis_index("c")
    s = lax.axis_index("s")
    wid = c * NS + s
    base = wid * E_PER_W
    lane = lax.iota(jnp.int32, L)
    zero16 = jnp.zeros((L,), jnp.float32)
    one16 = jnp.ones((L,), jnp.int32)

    def zero_body(i, _):
      for u in range(16):
        acc_v[pl.ds(i * (16 * L) + u * L, L)] = zero16
      return 0

    lax.fori_loop(0, ACC_FLAT // (16 * L), zero_body, 0)
    for u in range(ACC_FLAT % (16 * L) // L):
      acc_v[pl.ds((ACC_FLAT // (16 * L)) * 16 * L + u * L, L)] = zero16

    def zbmp_body(i, _):
      for u in range(10):
        bmp_v[pl.ds(i * (10 * L) + u * L, L)] = jnp.zeros((L,), jnp.int32)
      return 0

    lax.fori_loop(0, BMP_WORDS // (10 * L), zbmp_body, 0)

    pltpu.sync_copy(dst_hbm.at[pl.ds(base, E_PER_W)], dst_v)
    pltpu.sync_copy(src_hbm.at[pl.ds(base, E_PER_W)], src_v)
    pltpu.sync_copy(tgt_hbm, tgt_v)

    for v in range(N_GRAPHS // L):
      idx = lane + v * L
      t = tgt_v[pl.ds(v * L, L)]
      prev = plsc.load_gather(tgt_v, [jnp.maximum(idx - 1, 0)])
      fresh = (t != prev) | (idx == 0)
      bit = jnp.left_shift(one16, jnp.bitwise_and(t, 31))
      plsc.addupdate_scatter(bmp_v, [jnp.right_shift(t, 5)], bit, mask=fresh)

    def scan_body(i, cntv):
      upd = cntv
      for u in range(2):
        off = (2 * i + u) * L
        dstv = dst_v[pl.ds(off, L)]
        srcv = src_v[pl.ds(off, L)]
        word = plsc.load_gather(bmp_v, [jnp.right_shift(dstv, 5)])
        m = jnp.bitwise_and(
            jnp.right_shift(word, jnp.bitwise_and(dstv, 31)), 1) == 1
        pos = upd + plsc.cumsum(m.astype(jnp.int32)) - 1
        plsc.store_scatter(csrc_v, [pos], srcv, mask=m)
        plsc.store_scatter(cdst_v, [pos], dstv, mask=m)
        upd = upd + plsc.all_reduce_population_count(m)
      return upd

    cntv = lax.fori_loop(0, NVREG // 2, scan_body, jnp.zeros((L,), jnp.int32))
    csrc_v[pl.ds(E_PER_W, L)] = cntv
    cnt = csrc_v[pl.ds(E_PER_W, L)][0]

    def acc_body(j, _):
      k0 = j * L
      valid = lane < (cnt - k0)
      idxv = jnp.where(valid, csrc_v[pl.ds(k0, L)], 0)
      dstv = cdst_v[pl.ds(k0, L)]
      lo = jnp.zeros((L,), jnp.int32)
      for h in (64, 32, 16, 8, 4, 2, 1):
        tv = plsc.load_gather(tgt_v, [lo + (h - 1)])
        lo = jnp.where(tv < dstv, lo + h, lo)
      slot16_v[...] = jnp.where(valid, lo, N_GRAPHS)
      pltpu.async_copy(x_hbm.at[idxv], rows_v, sem).wait()
      slotvec = slot16_v[...]
      for r in range(L):
        slot_s = slotvec[r]

        @pl.when(slot_s < N_GRAPHS)
        def _():
          base = slot_s * D
          for col in range(D // L):
            plsc.addupdate(acc_v.at[pl.ds(base + col * L, L)],
                           rows_v[r, pl.ds(col * L, L)])
      return 0

    lax.fori_loop(0, (cnt + (L - 1)) // L, acc_body, 0)

    pltpu.sync_copy(acc_v.at[pl.ds(0, N_GRAPHS * D)], out_hbm.at[wid])

  return k(x, src, dst, targets)


def _tc_gram(partials, dst_r, tgt_col, tgt_row, x_label, scale_half):
  n_rows = dst_r.shape[0]

  def body(part_ref, dst_ref, tcol_ref, trow_ref, xl_ref, sh_ref, g_ref,
           xcat_ref):
    tcol = tcol_ref[...]

    def cnt_body(i, cnt):
      row = dst_ref[pl.ds(i, 1), :]
      eq = (tcol == row).astype(jnp.float32)
      return cnt + jnp.sum(eq, axis=1, keepdims=True)

    cnt = lax.fori_loop(0, n_rows, cnt_body,
                        jnp.zeros((N_GRAPHS, 1), jnp.float32))
    sums = jnp.sum(part_ref[...], axis=0)
    mean = sums / jnp.maximum(cnt, 1.0)

    eq2 = tcol == trow_ref[...]
    iota_col = lax.broadcasted_iota(jnp.int32, (N_GRAPHS, N_GRAPHS), 1)
    s_g = jnp.min(jnp.where(eq2, iota_col, N_GRAPHS), axis=1, keepdims=True)
    dmat = (iota_col == s_g).astype(jnp.float32)
    mean_full = lax.dot_general(dmat, mean, (((1,), (0,)), ((), ())),
                                preferred_element_type=jnp.float32,
                                precision=lax.Precision.HIGHEST)

    xcat_ref[pl.ds(0, N_GRAPHS), :] = mean_full
    xcat_ref[pl.ds(N_GRAPHS, N_LABELS), :] = xl_ref[...]
    xc = xcat_ref[...]
    nrm = jnp.sqrt(jnp.sum(xc * xc, axis=1, keepdims=True))
    xn = (xc / jnp.maximum(nrm, 1e-8)) * sh_ref[...]
    g_ref[...] = lax.dot_general(xn, xn, (((1,), (1,)), ((), ())),
                                 preferred_element_type=jnp.float32,
                                 precision=lax.Precision.HIGHEST)

  return pl.pallas_call(
      body,
      out_shape=jax.ShapeDtypeStruct((NCAT, NCAT), jnp.float32),
      scratch_shapes=[pltpu.VMEM((NCAT, D), jnp.float32)],
  )(partials, dst_r, tgt_col, tgt_row, x_label, scale_half)


def _sc_decode(g16, mg0, mg1):
  @functools.partial(
      pl.kernel,
      out_type=jax.ShapeDtypeStruct((M_EDGES,), jnp.float32),
      mesh=_mesh(),
      compiler_params=pltpu.CompilerParams(needs_layout_passes=False),
      scratch_types=[
          pltpu.VMEM((M_PER_W,), jnp.int32),
          pltpu.VMEM((M_PER_W,), jnp.int32),
          pltpu.VMEM((128,), jnp.int32),
          pltpu.VMEM((128, 128), jnp.float32),
          pltpu.VMEM((M_PER_W,), jnp.float32),
          pltpu.SemaphoreType.DMA,
      ],
  )
  def k(g_hbm, mg0_hbm, mg1_hbm, out_hbm,
        mg0_v, mg1_v, q_v, rows_v, out_v, sem):
    c = lax.axis_index("c")
    s = lax.axis_index("s")
    wid = c * NS + s
    base = wid * M_PER_W
    lane = lax.iota(jnp.int32, L)

    pltpu.sync_copy(mg0_hbm.at[pl.ds(base, M_PER_W)], mg0_v)
    pltpu.sync_copy(mg1_hbm.at[pl.ds(base, M_PER_W)], mg1_v)

    for sub in range(M_PER_W // 128):
      for v in range(128 // L):
        off = sub * 128 + v * L
        flat = mg0_v[pl.ds(off, L)] * NCAT + mg1_v[pl.ds(off, L)]
        q_v[pl.ds(v * L, L)] = jnp.right_shift(flat, 7)
      pltpu.async_copy(g_hbm.at[q_v], rows_v, sem).wait()
      for v in range(128 // L):
        off = sub * 128 + v * L
        flat = mg0_v[pl.ds(off, L)] * NCAT + mg1_v[pl.ds(off, L)]
        r = jnp.bitwise_and(flat, 127)
        out_v[pl.ds(off, L)] = plsc.load_gather(rows_v, [lane + v * L, r])
    pltpu.sync_copy(out_v, out_hbm.at[pl.ds(base, M_PER_W)])

  return k(g16, mg0, mg1)


def kernel(x, supernode, ptr, edge_index_supernode, x_label, y_true_matrix,
           metagraph_edge_index, metagraph_edge_attr, query_set_mask,
           logit_scale):
  src = edge_index_supernode[0]
  dst = edge_index_supernode[1]
  targets = supernode + ptr[:N_GRAPHS]

  partials = _sc_accumulate(x, src, dst, targets).reshape(NW, N_GRAPHS, D)

  scale_half = jnp.exp(logit_scale * 0.5).reshape(1, 1)
  g = _tc_gram(partials, dst.reshape(-1, 1024), targets.reshape(N_GRAPHS, 1),
               targets.reshape(1, N_GRAPHS), x_label, scale_half)

  decoded = _sc_decode(g.reshape(-1, 128), metagraph_edge_index[0],
                       metagraph_edge_index[1])
  y_pred = decoded.reshape(N_GRAPHS, N_LABELS)
  return (y_true_matrix, y_pred)
